# Initial kernel scaffold; baseline (speedup 1.0000x reference)
import jax
import jax.numpy as jnp
from jax.experimental import pallas as pl


def kernel(img, reg_lab):
    raise NotImplementedError("write your pallas kernel here")


# speedup vs baseline: 36.7789x; 36.7789x over previous
import functools

import jax
import jax.numpy as jnp
from jax import lax
from jax.experimental import pallas as pl
from jax.experimental.pallas import tpu as pltpu
from jax.experimental.pallas import tpu_sc as plsc

B = 4
C = 3
H = 384
W = 384
S = 200
CBINS = 25
TBINS = 10
D = 4
P = H * W
IMG_SIZE = float(P)

CH = 9216
NCHUNK = P // CH
CSIZE = S * C * CBINS
TSIZE = S * C * D * TBINS
RSIZE = H * S
HIST_MAX = RSIZE


def _shr(x, dy):
    z = jnp.zeros((1, W), jnp.float32)
    if dy == 1:
        return jnp.concatenate([x[1:, :], z], axis=0)
    return jnp.concatenate([z, x[:-1, :]], axis=0)


def _shc(x, dx):
    z = jnp.zeros((H, 1), jnp.float32)
    if dx == 1:
        return jnp.concatenate([x[:, 1:], z], axis=1)
    return jnp.concatenate([z, x[:, :-1]], axis=1)


def _stage1_body(img_ref, lab_ref, ocol_ref, otex_ref, orow_ref, ocol2_ref):
    c = pl.program_id(1)
    lab = lab_ref[0]
    p = img_ref[0, 0] * 255.0

    cbin = (p * jnp.float32((CBINS - 1) / 255.0)).astype(jnp.int32)
    ocol_ref[0, 0] = lab * (C * CBINS) + c * CBINS + cbin

    xm = _shr(p, -1)
    xp = _shr(p, 1)
    s = xm + xp
    d = xp - xm
    gx = 3.0 * (_shc(s, 1) - _shc(s, -1)) + 10.0 * (_shc(p, 1) + _shc(p, -1))
    gy = 3.0 * (_shc(d, 1) + _shc(d, -1)) + 10.0 * d

    derivs = (jnp.maximum(gx, 0.0), jnp.maximum(gy, 0.0),
              jnp.minimum(gx, 0.0), jnp.minimum(gy, 0.0))
    for di, dv in enumerate(derivs):
        mn = jnp.min(dv)
        mx = jnp.max(dv)
        norm = (dv - mn) / (mx - mn)
        tbin = jnp.clip((norm * jnp.float32(TBINS - 1)).astype(jnp.int32),
                        0, TBINS - 1)
        otex_ref[0, di] = lab * (C * D * TBINS) + (c * D + di) * TBINS + tbin

    yy = lax.broadcasted_iota(jnp.int32, (H, W), 0)
    xx = lax.broadcasted_iota(jnp.int32, (H, W), 1)
    orow_ref[0] = yy * S + lab
    ocol2_ref[0] = xx * S + lab


def _stage1(img, reg_lab):
    return pl.pallas_call(
        _stage1_body,
        grid=(B, C),
        in_specs=[
            pl.BlockSpec((1, 1, H, W), lambda b, c: (b, c, 0, 0)),
            pl.BlockSpec((1, H, W), lambda b, c: (b, 0, 0)),
        ],
        out_specs=[
            pl.BlockSpec((1, 1, H, W), lambda b, c: (b, c, 0, 0)),
            pl.BlockSpec((1, D, H, W), lambda b, c: (b, c, 0, 0)),
            pl.BlockSpec((1, H, W), lambda b, c: (b, 0, 0)),
            pl.BlockSpec((1, H, W), lambda b, c: (b, 0, 0)),
        ],
        out_shape=[
            jax.ShapeDtypeStruct((B, C, H, W), jnp.int32),
            jax.ShapeDtypeStruct((B, C * D, H, W), jnp.int32),
            jax.ShapeDtypeStruct((B, H, W), jnp.int32),
            jax.ShapeDtypeStruct((B, H, W), jnp.int32),
        ],
    )(img, reg_lab)


def _zero_hist(hist, size):
    n16 = -(-size // 16)
    full, rem = n16 // 8, n16 % 8
    z = jnp.zeros((16,), jnp.float32)

    def zb(i, _):
        for u in range(8):
            hist[pl.ds(i * 128 + u * 16, 16)] = z
        return 0

    lax.fori_loop(0, full, zb, 0)
    for u in range(rem):
        hist[pl.ds(full * 128 + u * 16, 16)] = z


def _hist_task(in_hbm, in_off, out_ref, size, buf, hist, sems):
    _zero_hist(hist, size)
    ones = jnp.ones((16,), jnp.float32)
    handles = [None, None]
    handles[0] = pltpu.async_copy(
        in_hbm.at[pl.ds(in_off, CH)], buf.at[0], sems[0])
    for ci in range(NCHUNK):
        cur = ci & 1
        handles[cur].wait()
        if ci + 1 < NCHUNK:
            handles[1 - cur] = pltpu.async_copy(
                in_hbm.at[pl.ds(in_off + (ci + 1) * CH, CH)],
                buf.at[1 - cur], sems[1 - cur])

        def sb(i, _):
            for u in range(8):
                idx = buf[cur, pl.ds(i * 128 + u * 16, 16)]
                plsc.addupdate_scatter(hist, [idx], ones)
            return 0

        lax.fori_loop(0, CH // 128, sb, 0)
    pltpu.sync_copy(hist.at[pl.ds(0, size)], out_ref)


def _sc_body(col_hbm, tex_hbm, row_hbm, cxx_hbm,
             ccnt, tcnt, rcnt, xcnt, buf, hist, sa, sb):
    w = lax.axis_index("c") * 16 + lax.axis_index("s")
    sems = (sa, sb)

    @pl.when(w < 12)
    def _():
        _hist_task(col_hbm, w * P, ccnt.at[w], CSIZE, buf, hist, sems)

    tta = jnp.where(w < 12, w + 20, w - 12)
    _hist_task(tex_hbm, tta * P, tcnt.at[tta], TSIZE, buf, hist, sems)

    @pl.when((w >= 12) & (w <= 27))
    def _():
        ttb = w + 20
        _hist_task(tex_hbm, ttb * P, tcnt.at[ttb], TSIZE, buf, hist, sems)

    @pl.when(w >= 28)
    def _():
        rb = w - 28
        _hist_task(row_hbm, rb * P, rcnt.at[rb], RSIZE, buf, hist, sems)

    @pl.when((w & 7) == 0)
    def _():
        cb = lax.shift_right_logical(w, 3)
        _hist_task(cxx_hbm, cb * P, xcnt.at[cb], RSIZE, buf, hist, sems)


def _stage2(idx_color, idx_tex, idx_row, idx_col):
    mesh = plsc.VectorSubcoreMesh(core_axis_name="c", subcore_axis_name="s",
                                  num_cores=2, num_subcores=16)
    f = functools.partial(
        pl.kernel,
        out_type=[
            jax.ShapeDtypeStruct((B * C, CSIZE), jnp.float32),
            jax.ShapeDtypeStruct((B * C * D, TSIZE), jnp.float32),
            jax.ShapeDtypeStruct((B, RSIZE), jnp.float32),
            jax.ShapeDtypeStruct((B, RSIZE), jnp.float32),
        ],
        mesh=mesh,
        compiler_params=pltpu.CompilerParams(needs_layout_passes=False,
                                             use_tc_tiling_on_sc=False),
        scratch_types=[
            pltpu.VMEM((2, CH), jnp.int32),
            pltpu.VMEM((HIST_MAX,), jnp.float32),
            pltpu.SemaphoreType.DMA,
            pltpu.SemaphoreType.DMA,
        ],
    )(_sc_body)
    return f(idx_color.reshape(-1), idx_tex.reshape(-1),
             idx_row.reshape(-1), idx_col.reshape(-1))


def _stage3_body(ccnt_ref, tcnt_ref, rc_ref, xc_ref, out_ref):
    csum = ccnt_ref[0, 0] + ccnt_ref[0, 1] + ccnt_ref[0, 2]
    t = tcnt_ref[0]
    tsum = t[0]
    for i in range(1, C * D):
        tsum = tsum + t[i]

    sizes3_col = jnp.sum(csum, axis=1, keepdims=True)
    hc = csum / sizes3_col
    ht = tsum / (4.0 * sizes3_col)
    hct = jnp.transpose(hc)
    htt = jnp.transpose(ht)

    rc = rc_ref[0]
    xc = xc_ref[0]
    size_row = jnp.sum(rc, axis=0, keepdims=True)
    yi = lax.broadcasted_iota(jnp.int32, (H, S), 0).astype(jnp.float32)
    big = jnp.float32(2.0 ** 30)
    ymin_r = jnp.min(jnp.where(rc > 0, yi, big), axis=0, keepdims=True)
    ymax_r = jnp.max(jnp.where(rc > 0, yi, -big), axis=0, keepdims=True)
    xmin_r = jnp.min(jnp.where(xc > 0, yi, big), axis=0, keepdims=True)
    xmax_r = jnp.max(jnp.where(xc > 0, yi, -big), axis=0, keepdims=True)

    v = jnp.concatenate(
        [size_row, ymin_r, ymax_r, xmin_r, xmax_r,
         jnp.zeros((3, S), jnp.float32)], axis=0)
    vt = jnp.transpose(v)
    size_col = vt[:, 0:1]
    ymin_c = vt[:, 1:2]
    ymax_c = vt[:, 2:3]
    xmin_c = vt[:, 3:4]
    xmax_c = vt[:, 4:5]

    inv = jnp.float32(1.0 / IMG_SIZE)
    acc = 1.0 - (size_col + size_row) * inv
    x1 = jnp.minimum(xmin_c, xmin_r)
    y1 = jnp.minimum(ymin_c, ymin_r)
    x2 = jnp.maximum(xmax_c, xmax_r)
    y2 = jnp.maximum(ymax_c, ymax_r)
    merged = (x2 - x1 + 1.0) * (y2 - y1 + 1.0)
    acc = acc + (1.0 - (merged - size_col - size_row) * inv)

    for k in range(C * CBINS):
        acc = acc + jnp.minimum(hc[:, k:k + 1], hct[k:k + 1, :])
    for k in range(C * D * TBINS):
        acc = acc + jnp.minimum(ht[:, k:k + 1], htt[k:k + 1, :])
    out_ref[0] = acc


def _stage3(ccnt, tcnt, rcnt, xcnt):
    return pl.pallas_call(
        _stage3_body,
        grid=(B,),
        in_specs=[
            pl.BlockSpec((1, C, S, C * CBINS), lambda b: (b, 0, 0, 0)),
            pl.BlockSpec((1, C * D, S, C * D * TBINS), lambda b: (b, 0, 0, 0)),
            pl.BlockSpec((1, H, S), lambda b: (b, 0, 0)),
            pl.BlockSpec((1, W, S), lambda b: (b, 0, 0)),
        ],
        out_specs=pl.BlockSpec((1, S, S), lambda b: (b, 0, 0)),
        out_shape=jax.ShapeDtypeStruct((B, S, S), jnp.float32),
    )(ccnt, tcnt, rcnt, xcnt)


def kernel(img, reg_lab):
    idx_color, idx_tex, idx_row, idx_col = _stage1(img, reg_lab)
    ccnt, tcnt, rcnt, xcnt = _stage2(idx_color, idx_tex, idx_row, idx_col)
    return _stage3(
        ccnt.reshape(B, C, S, C * CBINS),
        tcnt.reshape(B, C * D, S, C * D * TBINS),
        rcnt.reshape(B, H, S),
        xcnt.reshape(B, W, S),
    )


# speedup vs baseline: 60.4652x; 1.6440x over previous
import functools

import jax
import jax.numpy as jnp
from jax import lax
from jax.experimental import pallas as pl
from jax.experimental.pallas import tpu as pltpu
from jax.experimental.pallas import tpu_sc as plsc

B = 4
C = 3
H = 384
W = 384
S = 200
CBINS = 25
TBINS = 10
D = 4
P = H * W
IMG_SIZE = float(P)

HALF = P // 2
CH = 4608
NCH_HALF = HALF // CH
NCH_FULL = P // CH
P0SZ = S * CBINS * TBINS
P1SZ = S * TBINS * TBINS
P2SZ = S * TBINS
PAIRSZ = P0SZ + P1SZ + P2SZ
RSIZE = H * S


def _shr(x, dy):
    z = jnp.zeros((1, W), jnp.float32)
    if dy == 1:
        return jnp.concatenate([x[1:, :], z], axis=0)
    return jnp.concatenate([z, x[:-1, :]], axis=0)


def _shc(x, dx):
    z = jnp.zeros((H, 1), jnp.float32)
    if dx == 1:
        return jnp.concatenate([x[:, 1:], z], axis=1)
    return jnp.concatenate([z, x[:, :-1]], axis=1)


def _stage1_body(img_ref, lab_ref, oa_ref, ob_ref, orow_ref, ocol_ref):
    lab = lab_ref[0]
    p = img_ref[0, 0] * 255.0

    cbin = (p * jnp.float32((CBINS - 1) / 255.0)).astype(jnp.int32)

    xm = _shr(p, -1)
    xp = _shr(p, 1)
    s = xm + xp
    d = xp - xm
    gx = 3.0 * (_shc(s, 1) - _shc(s, -1)) + 10.0 * (_shc(p, 1) + _shc(p, -1))
    gy = 3.0 * (_shc(d, 1) + _shc(d, -1)) + 10.0 * d

    tb = []
    for dv in (jnp.maximum(gx, 0.0), jnp.maximum(gy, 0.0),
               jnp.minimum(gx, 0.0), jnp.minimum(gy, 0.0)):
        mn = jnp.min(dv)
        mx = jnp.max(dv)
        norm = (dv - mn) / (mx - mn)
        tb.append(jnp.clip((norm * jnp.float32(TBINS - 1)).astype(jnp.int32),
                           0, TBINS - 1))

    oa_ref[0, 0] = lab * (CBINS * TBINS) + cbin * TBINS + tb[0]
    ob_ref[0, 0] = ((lab * (TBINS * TBINS) + tb[1] * TBINS + tb[2])
                    | ((lab * TBINS + tb[3]) << 17))

    yy = lax.broadcasted_iota(jnp.int32, (H, W), 0)
    xx = lax.broadcasted_iota(jnp.int32, (H, W), 1)
    orow_ref[0] = yy * S + lab
    ocol_ref[0] = xx * S + lab


def _stage1(img, reg_lab):
    return pl.pallas_call(
        _stage1_body,
        grid=(B, C),
        in_specs=[
            pl.BlockSpec((1, 1, H, W), lambda b, c: (b, c, 0, 0)),
            pl.BlockSpec((1, H, W), lambda b, c: (b, 0, 0)),
        ],
        out_specs=[
            pl.BlockSpec((1, 1, H, W), lambda b, c: (b, c, 0, 0)),
            pl.BlockSpec((1, 1, H, W), lambda b, c: (b, c, 0, 0)),
            pl.BlockSpec((1, H, W), lambda b, c: (b, 0, 0)),
            pl.BlockSpec((1, H, W), lambda b, c: (b, 0, 0)),
        ],
        out_shape=[
            jax.ShapeDtypeStruct((B, C, H, W), jnp.int32),
            jax.ShapeDtypeStruct((B, C, H, W), jnp.int32),
            jax.ShapeDtypeStruct((B, H, W), jnp.int32),
            jax.ShapeDtypeStruct((B, H, W), jnp.int32),
        ],
    )(img, reg_lab)


def _zero_hist(hist, size):
    n16 = -(-size // 16)
    full, rem = n16 // 8, n16 % 8
    z = jnp.zeros((16,), jnp.float32)

    def zb(i, _):
        for u in range(8):
            hist[pl.ds(i * 128 + u * 16, 16)] = z
        return 0

    lax.fori_loop(0, full, zb, 0)
    for u in range(rem):
        hist[pl.ds(full * 128 + u * 16, 16)] = z


def _pair_task(a_hbm, b_hbm, off, out_ref, bufa, bufb, hist, sems):
    _zero_hist(hist, PAIRSZ)
    ones = jnp.ones((16,), jnp.float32)
    m15 = jnp.int32(0x7FFF)
    k1 = jnp.int32(P0SZ)
    k2 = jnp.int32(P0SZ + P1SZ)
    out0, out1, out2 = out_ref
    hs = [None, None]

    def start(ci, buf_i):
        hs[buf_i] = (
            pltpu.async_copy(a_hbm.at[pl.ds(off + ci * CH, CH)],
                             bufa.at[buf_i], sems[0][buf_i]),
            pltpu.async_copy(b_hbm.at[pl.ds(off + ci * CH, CH)],
                             bufb.at[buf_i], sems[1][buf_i]),
        )

    start(0, 0)
    for ci in range(NCH_HALF):
        cur = ci & 1
        hs[cur][0].wait()
        hs[cur][1].wait()
        if ci + 1 < NCH_HALF:
            start(ci + 1, 1 - cur)

        def sb(i, _):
            for u in range(4):
                st = i * 64 + u * 16
                wa = bufa[cur, pl.ds(st, 16)]
                wb = bufb[cur, pl.ds(st, 16)]
                plsc.addupdate_scatter(hist, [wa], ones)
                plsc.addupdate_scatter(hist, [(wb & m15) + k1], ones)
                plsc.addupdate_scatter(
                    hist, [lax.shift_right_logical(wb, 17) + k2], ones)
            return 0

        lax.fori_loop(0, CH // 64, sb, 0)
    pltpu.sync_copy(hist.at[pl.ds(0, P0SZ)], out0)
    pltpu.sync_copy(hist.at[pl.ds(P0SZ, P1SZ)], out1)
    pltpu.sync_copy(hist.at[pl.ds(P0SZ + P1SZ, P2SZ)], out2)


def _count_task(in_hbm, off, out_ref, bufa, hist, sems):
    _zero_hist(hist, RSIZE)
    ones = jnp.ones((16,), jnp.float32)
    hs = [None, None]
    hs[0] = pltpu.async_copy(in_hbm.at[pl.ds(off, CH)], bufa.at[0], sems[0][0])
    for ci in range(NCH_FULL):
        cur = ci & 1
        hs[cur].wait()
        if ci + 1 < NCH_FULL:
            hs[1 - cur] = pltpu.async_copy(
                in_hbm.at[pl.ds(off + (ci + 1) * CH, CH)],
                bufa.at[1 - cur], sems[0][1 - cur])

        def sb(i, _):
            for u in range(4):
                idx = bufa[cur, pl.ds(i * 64 + u * 16, 16)]
                plsc.addupdate_scatter(hist, [idx], ones)
            return 0

        lax.fori_loop(0, CH // 64, sb, 0)
    pltpu.sync_copy(hist.at[pl.ds(0, RSIZE)], out_ref)


def _sc_body(a_hbm, b_hbm, row_hbm, col_hbm,
             p0out, p1out, p2out, rcnt, xcnt,
             bufa, bufb, hist, s0a, s0b, s1a, s1b):
    w = lax.axis_index("c") * 16 + lax.axis_index("s")
    sems = ((s0a, s0b), (s1a, s1b))

    @pl.when(w < 24)
    def _():
        bc = lax.shift_right_logical(w, 1)
        half = w & 1
        off = bc * P + half * HALF
        _pair_task(a_hbm, b_hbm, off,
                   (p0out.at[w], p1out.at[w], p2out.at[w]),
                   bufa, bufb, hist, sems)

    @pl.when((w >= 24) & (w < 28))
    def _():
        b = w - 24
        _count_task(row_hbm, b * P, rcnt.at[b], bufa, hist, sems)

    @pl.when(w >= 28)
    def _():
        b = w - 28
        _count_task(col_hbm, b * P, xcnt.at[b], bufa, hist, sems)


def _stage2(a, bb, idx_row, idx_col):
    mesh = plsc.VectorSubcoreMesh(core_axis_name="c", subcore_axis_name="s",
                                  num_cores=2, num_subcores=16)
    f = functools.partial(
        pl.kernel,
        out_type=[
            jax.ShapeDtypeStruct((B * C * 2, P0SZ), jnp.float32),
            jax.ShapeDtypeStruct((B * C * 2, P1SZ), jnp.float32),
            jax.ShapeDtypeStruct((B * C * 2, P2SZ), jnp.float32),
            jax.ShapeDtypeStruct((B, RSIZE), jnp.float32),
            jax.ShapeDtypeStruct((B, RSIZE), jnp.float32),
        ],
        mesh=mesh,
        compiler_params=pltpu.CompilerParams(needs_layout_passes=False,
                                             use_tc_tiling_on_sc=False),
        scratch_types=[
            pltpu.VMEM((2, CH), jnp.int32),
            pltpu.VMEM((2, CH), jnp.int32),
            pltpu.VMEM((PAIRSZ,), jnp.float32),
            pltpu.SemaphoreType.DMA,
            pltpu.SemaphoreType.DMA,
            pltpu.SemaphoreType.DMA,
            pltpu.SemaphoreType.DMA,
        ],
    )(_sc_body)
    return f(a.reshape(-1), bb.reshape(-1),
             idx_row.reshape(-1), idx_col.reshape(-1))


def _sel_major(n, m):
    i = jnp.arange(n * m) // m
    return (i[:, None] == jnp.arange(n)[None, :]).astype(jnp.float32)


def _sel_minor(n, m):
    j = jnp.arange(n * m) % m
    return (j[:, None] == jnp.arange(m)[None, :]).astype(jnp.float32)


def _stage3_body(p0_ref, p1_ref, p2_ref, rc_ref, xc_ref,
                 ma_ref, mb_ref, mc_ref, md_ref, out_ref):
    cparts = []
    tparts = []
    for c in range(C):
        p0 = p0_ref[0, c, 0] + p0_ref[0, c, 1]
        p1 = p1_ref[0, c, 0] + p1_ref[0, c, 1]
        p2 = p2_ref[0, c, 0] + p2_ref[0, c, 1]
        cc = jnp.dot(p0, ma_ref[...], preferred_element_type=jnp.float32)
        t0 = jnp.dot(p0, mb_ref[...], preferred_element_type=jnp.float32)
        t1 = jnp.dot(p1, mc_ref[...], preferred_element_type=jnp.float32)
        t2 = jnp.dot(p1, md_ref[...], preferred_element_type=jnp.float32)
        cparts.append(cc)
        tparts.extend([t0, t1, t2, p2])
    csum = jnp.concatenate(cparts, axis=1)
    tsum = jnp.concatenate(tparts, axis=1)

    sizes3_col = jnp.sum(csum, axis=1, keepdims=True)
    hc = csum / sizes3_col
    ht = tsum / (4.0 * sizes3_col)
    hct = jnp.transpose(hc)
    htt = jnp.transpose(ht)

    rc = rc_ref[0]
    xc = xc_ref[0]
    size_row = jnp.sum(rc, axis=0, keepdims=True)
    yi = lax.broadcasted_iota(jnp.int32, (H, S), 0).astype(jnp.float32)
    big = jnp.float32(2.0 ** 30)
    ymin_r = jnp.min(jnp.where(rc > 0, yi, big), axis=0, keepdims=True)
    ymax_r = jnp.max(jnp.where(rc > 0, yi, -big), axis=0, keepdims=True)
    xmin_r = jnp.min(jnp.where(xc > 0, yi, big), axis=0, keepdims=True)
    xmax_r = jnp.max(jnp.where(xc > 0, yi, -big), axis=0, keepdims=True)

    v = jnp.concatenate(
        [size_row, ymin_r, ymax_r, xmin_r, xmax_r,
         jnp.zeros((3, S), jnp.float32)], axis=0)
    vt = jnp.transpose(v)
    size_col = vt[:, 0:1]
    ymin_c = vt[:, 1:2]
    ymax_c = vt[:, 2:3]
    xmin_c = vt[:, 3:4]
    xmax_c = vt[:, 4:5]

    inv = jnp.float32(1.0 / IMG_SIZE)
    acc = 1.0 - (size_col + size_row) * inv
    x1 = jnp.minimum(xmin_c, xmin_r)
    y1 = jnp.minimum(ymin_c, ymin_r)
    x2 = jnp.maximum(xmax_c, xmax_r)
    y2 = jnp.maximum(ymax_c, ymax_r)
    merged = (x2 - x1 + 1.0) * (y2 - y1 + 1.0)
    acc = acc + (1.0 - (merged - size_col - size_row) * inv)

    for k in range(C * CBINS):
        acc = acc + jnp.minimum(hc[:, k:k + 1], hct[k:k + 1, :])
    for k in range(C * D * TBINS):
        acc = acc + jnp.minimum(ht[:, k:k + 1], htt[k:k + 1, :])
    out_ref[0] = acc


def _stage3(p0, p1, p2, rcnt, xcnt):
    ma = _sel_major(CBINS, TBINS)
    mb = _sel_minor(CBINS, TBINS)
    mc = _sel_major(TBINS, TBINS)
    md = _sel_minor(TBINS, TBINS)
    return pl.pallas_call(
        _stage3_body,
        grid=(B,),
        in_specs=[
            pl.BlockSpec((1, C, 2, S, CBINS * TBINS),
                         lambda b: (b, 0, 0, 0, 0)),
            pl.BlockSpec((1, C, 2, S, TBINS * TBINS),
                         lambda b: (b, 0, 0, 0, 0)),
            pl.BlockSpec((1, C, 2, S, TBINS), lambda b: (b, 0, 0, 0, 0)),
            pl.BlockSpec((1, H, S), lambda b: (b, 0, 0)),
            pl.BlockSpec((1, W, S), lambda b: (b, 0, 0)),
            pl.BlockSpec((CBINS * TBINS, CBINS), lambda b: (0, 0)),
            pl.BlockSpec((CBINS * TBINS, TBINS), lambda b: (0, 0)),
            pl.BlockSpec((TBINS * TBINS, TBINS), lambda b: (0, 0)),
            pl.BlockSpec((TBINS * TBINS, TBINS), lambda b: (0, 0)),
        ],
        out_specs=pl.BlockSpec((1, S, S), lambda b: (b, 0, 0)),
        out_shape=jax.ShapeDtypeStruct((B, S, S), jnp.float32),
    )(p0, p1, p2, rcnt, xcnt, ma, mb, mc, md)


def kernel(img, reg_lab):
    a, bb, idx_row, idx_col = _stage1(img, reg_lab)
    p0, p1, p2, rcnt, xcnt = _stage2(a, bb, idx_row, idx_col)
    return _stage3(p0.reshape(B, C, 2, S, CBINS * TBINS),
                   p1.reshape(B, C, 2, S, TBINS * TBINS),
                   p2.reshape(B, C, 2, S, TBINS),
                   rcnt.reshape(B, H, S),
                   xcnt.reshape(B, W, S))
